# baseline (speedup 1.0000x reference)
import functools

import jax
import jax.numpy as jnp
from jax import lax
from jax.experimental import pallas as pl
from jax.experimental.pallas import tpu as pltpu
from jax.experimental.pallas import tpu_sc as plsc

_NC, _NS, _L = 2, 16, 16
_NW = _NC * _NS
_S = 49
_CHUNK = 3072
_G = 16


def _cdiv(a, b):
    return (a + b - 1) // b


def _sc_split(img2d, widx_in, widx_tg, n_grp_in, n_grp_tg):
    ni_in = widx_in.shape[1] // _G
    ni_tg = widx_tg.shape[1] // _G
    mesh = plsc.VectorSubcoreMesh(core_axis_name="c", subcore_axis_name="s")

    @functools.partial(
        pl.kernel,
        out_type=[
            jax.ShapeDtypeStruct((n_grp_in * _G, _CHUNK), jnp.float32),
            jax.ShapeDtypeStruct((n_grp_tg * _G, _CHUNK), jnp.float32),
        ],
        mesh=mesh,
        scratch_types=[
            pltpu.VMEM((ni_in * _G,), jnp.int32),
            pltpu.VMEM((ni_tg * _G,), jnp.int32),
            pltpu.VMEM((_G, _CHUNK), jnp.float32),
            pltpu.VMEM((_G, _CHUNK), jnp.float32),
            pltpu.SemaphoreType.DMA((2,)),
            pltpu.SemaphoreType.DMA((2,)),
        ],
    )
    def k(img_hbm, gi_hbm, gt_hbm, out_in, out_tg, iv_in, iv_tg, buf0, buf1,
          gsem, ssem):
        wid = lax.axis_index("s") * _NC + lax.axis_index("c")
        pltpu.sync_copy(gi_hbm.at[wid], iv_in)
        pltpu.sync_copy(gt_hbm.at[wid], iv_tg)
        bufs = (buf0, buf1)

        items = [(iv_in, i, out_in, n_grp_in) for i in range(ni_in)]
        items += [(iv_tg, i, out_tg, n_grp_tg) for i in range(ni_tg)]
        n = len(items)

        def start_gather(item, slot):
            iv, i, _, _ = item
            c = pltpu.make_async_copy(
                img_hbm.at[iv.at[pl.ds(i * _G, _G)]], bufs[slot],
                gsem.at[slot])
            c.start()
            return c

        def start_scatter(item, slot):
            _, i, out_ref, ng = item
            g = jnp.minimum(wid + i * _NW, ng - 1)
            c = pltpu.make_async_copy(
                bufs[slot], out_ref.at[pl.ds(g * _G, _G)], ssem.at[slot])
            c.start()
            return c

        gathers = [None] * n
        scatters = [None] * n
        gathers[0] = start_gather(items[0], 0)
        for i in range(n):
            p = i % 2
            gathers[i].wait()
            scatters[i] = start_scatter(items[i], p)
            if i + 1 < n:
                if i >= 1:
                    scatters[i - 1].wait()
                gathers[i + 1] = start_gather(items[i + 1], 1 - p)
        if n >= 2:
            scatters[n - 2].wait()
        scatters[n - 1].wait()

    return k(img2d, widx_in, widx_tg)


def _worker_groups(indices, B, V, n):
    base = (jnp.arange(B, dtype=jnp.int32)[:, None] * V + indices) * _S
    rows = base[:, :, None] + jnp.arange(_S, dtype=jnp.int32)[None, None, :]
    grp = rows.reshape(-1, _G)
    n_grp = grp.shape[0]
    niter = _cdiv(n_grp, _NW)
    pad = niter * _NW - n_grp
    grp = jnp.concatenate([grp, jnp.tile(grp[-1:], (pad, 1))], axis=0)
    widx = grp.reshape(niter, _NW, _G).transpose(1, 0, 2).reshape(_NW, niter * _G)
    return widx, n_grp


def kernel(image, context_indices, target_indices):
    B, V, C, H, W = image.shape
    n_in = context_indices.shape[1]
    n_tg = target_indices.shape[1]
    img2d = image.reshape(B * V * _S, _CHUNK)
    widx_in, n_grp_in = _worker_groups(context_indices, B, V, n_in)
    widx_tg, n_grp_tg = _worker_groups(target_indices, B, V, n_tg)
    out_in, out_tg = _sc_split(img2d, widx_in, widx_tg, n_grp_in, n_grp_tg)
    input_image = out_in.reshape(B, n_in, C, H, W)
    target_image = out_tg.reshape(B, n_tg, C, H, W)
    return (input_image, target_image, context_indices, target_indices)


# speedup vs baseline: 2.5815x; 2.5815x over previous
import jax
import jax.numpy as jnp
from jax.experimental import pallas as pl
from jax.experimental.pallas import tpu as pltpu


def _copy_body(idx_ref, in_ref, out_ref):
    out_ref[...] = in_ref[...]


def _gather_views(image, indices, n):
    B, V, C, H, W = image.shape
    return pl.pallas_call(
        _copy_body,
        grid_spec=pltpu.PrefetchScalarGridSpec(
            num_scalar_prefetch=1,
            grid=(B, n),
            in_specs=[pl.BlockSpec((1, 1, C, H, W),
                                   lambda b, v, idx: (b, idx[b, v], 0, 0, 0))],
            out_specs=pl.BlockSpec((1, 1, C, H, W),
                                   lambda b, v, idx: (b, v, 0, 0, 0)),
        ),
        out_shape=jax.ShapeDtypeStruct((B, n, C, H, W), image.dtype),
    )(indices, image)


def kernel(image, context_indices, target_indices):
    n_in = context_indices.shape[1]
    n_tg = target_indices.shape[1]
    input_image = _gather_views(image, context_indices, n_in)
    target_image = _gather_views(image, target_indices, n_tg)
    return (input_image, target_image, context_indices, target_indices)
